# baseline (speedup 1.0000x reference)
import jax
import jax.numpy as jnp
from jax import lax
from jax.experimental import pallas as pl
from jax.experimental.pallas import tpu as pltpu
from jax.experimental.pallas import tpu_sc as plsc

NUM_EMB = 1000000
DIM = 64
B = 16384
H = 50

NC = 2
NS = 16
NW = NC * NS

BGPW = (B // 128) // NW
N_UNITS = H * BGPW


def _emb_kernel(ids_hbm, w_hbm, out_hbm, idsv, idx0, idx1, rows0, rows1,
                tile0, tile1, isem, gsem0, gsem1, tsem0, tsem1):
    wid = lax.axis_index("s") * NC + lax.axis_index("c")
    idxb = (idx0, idx1)
    rows = (rows0, rows1)
    tile = (tile0, tile1)
    gsem = (gsem0, gsem1)
    tsem = (tsem0, tsem1)

    pltpu.async_copy(ids_hbm.at[:, pl.ds(wid * 512, 512)], idsv, isem).wait()

    lane = lax.iota(jnp.int32, 16)

    def prep_idx(u, p):
        h = u // BGPW
        bgl = u % BGPW
        for k in range(8):
            idxb[p][pl.ds(k * 16, 16)] = idsv[h, pl.ds(bgl * 128 + k * 16, 16)]

    def start_gather(p):
        pltpu.async_copy(w_hbm.at[idxb[p]], rows[p], gsem[p])

    def wait_gather(p):
        pltpu.make_async_copy(w_hbm.at[idxb[p]], rows[p], gsem[p]).wait()

    def transpose(p):
        @pl.loop(0, 128)
        def _row(b):
            base = jnp.full((16,), 0, jnp.int32) + b
            for k in range(4):
                v = rows[p][b, pl.ds(k * 16, 16)]
                plsc.store_scatter(tile[p], [(lane + (k * 16)) * 128 + base], v)

    def start_stores(u, p):
        h = u // BGPW
        bg = wid * BGPW + (u % BGPW)
        for dg in range(8):
            pltpu.async_copy(tile[p].at[pl.ds(dg * 1024, 1024)],
                             out_hbm.at[h, dg, bg], tsem[p])

    def wait_stores(u, p):
        h = u // BGPW
        bg = wid * BGPW + (u % BGPW)
        for dg in range(8):
            pltpu.make_async_copy(tile[p].at[pl.ds(dg * 1024, 1024)],
                                  out_hbm.at[h, dg, bg], tsem[p]).wait()

    def body(u, p, do_wait_store, do_gather):
        if do_gather:
            prep_idx(u + 1, 1 - p)
            start_gather(1 - p)
        wait_gather(p)
        if do_wait_store:
            wait_stores(u - 2, p)
        transpose(p)
        start_stores(u, p)

    prep_idx(0, 0)
    start_gather(0)
    body(0, 0, False, True)
    body(1, 1, False, True)

    @pl.loop(2, N_UNITS - 2, step=2)
    def _steady(u0):
        body(u0, 0, True, True)
        body(u0 + 1, 1, True, True)

    body(N_UNITS - 2, 0, True, True)
    body(N_UNITS - 1, 1, True, False)

    wait_stores(N_UNITS - 2, 0)
    wait_stores(N_UNITS - 1, 1)


@jax.jit
def _emb(ids_t, weight):
    mesh = plsc.VectorSubcoreMesh(
        core_axis_name="c", subcore_axis_name="s", num_cores=NC, num_subcores=NS
    )
    run = pl.kernel(
        _emb_kernel,
        out_type=jax.ShapeDtypeStruct((H, 8, 128, 1024), jnp.float32),
        mesh=mesh,
        scratch_types=[
            pltpu.VMEM((H, 512), jnp.int32),
            pltpu.VMEM((128,), jnp.int32),
            pltpu.VMEM((128,), jnp.int32),
            pltpu.VMEM((128, DIM), jnp.float32),
            pltpu.VMEM((128, DIM), jnp.float32),
            pltpu.VMEM((8192,), jnp.float32),
            pltpu.VMEM((8192,), jnp.float32),
        ] + [pltpu.SemaphoreType.DMA] * 5,
        compiler_params=pltpu.CompilerParams(use_tc_tiling_on_sc=False,
                                             needs_layout_passes=False),
    )
    return run(ids_t, weight)


def kernel(input_ids, weight):
    ids_t = input_ids.T
    out7 = _emb(ids_t, weight).reshape(H, 8, 128, 8, 128)
    return out7.transpose(2, 4, 0, 1, 3).reshape(B, H, DIM)


# speedup vs baseline: 1.0001x; 1.0001x over previous
import jax
import jax.numpy as jnp
from jax import lax
from jax.experimental import pallas as pl
from jax.experimental.pallas import tpu as pltpu
from jax.experimental.pallas import tpu_sc as plsc

NUM_EMB = 1000000
DIM = 64
B = 16384
H = 50

NC = 2
NS = 16
NW = NC * NS

BGPW = (B // 128) // NW
N_UNITS = H * BGPW


def _emb_kernel(ids_hbm, w_hbm, out_hbm, idsv, idx0, idx1, rows0, rows1,
                tile0, tile1, isem, gsem0, gsem1, tsem0, tsem1):
    wid = lax.axis_index("s") * NC + lax.axis_index("c")
    idxb = (idx0, idx1)
    rows = (rows0, rows1)
    tile = (tile0, tile1)
    gsem = (gsem0, gsem1)
    tsem = (tsem0, tsem1)

    pltpu.async_copy(ids_hbm.at[:, pl.ds(wid * 512, 512)], idsv, isem).wait()

    lane = lax.iota(jnp.int32, 16)

    def prep_idx(u, p):
        h = u // BGPW
        bgl = u % BGPW
        for k in range(8):
            idxb[p][pl.ds(k * 16, 16)] = idsv[h, pl.ds(bgl * 128 + k * 16, 16)]

    def start_gather(p):
        pltpu.async_copy(w_hbm.at[idxb[p]], rows[p], gsem[p])

    def wait_gather(p):
        pltpu.make_async_copy(w_hbm.at[idxb[p]], rows[p], gsem[p]).wait()

    sbase = [lane * 128 + k * 2048 for k in range(4)]

    def transpose(p):
        @pl.loop(0, 128, unroll=16)
        def _row(b):
            for k in range(4):
                v = rows[p][b, pl.ds(k * 16, 16)]
                plsc.store_scatter(tile[p], [sbase[k] + b], v)

    def start_stores(u, p):
        h = u // BGPW
        bg = wid * BGPW + (u % BGPW)
        for dg in range(8):
            pltpu.async_copy(tile[p].at[pl.ds(dg * 1024, 1024)],
                             out_hbm.at[h, dg, bg], tsem[p])

    def wait_stores(u, p):
        h = u // BGPW
        bg = wid * BGPW + (u % BGPW)
        for dg in range(8):
            pltpu.make_async_copy(tile[p].at[pl.ds(dg * 1024, 1024)],
                                  out_hbm.at[h, dg, bg], tsem[p]).wait()

    def body(u, p, do_wait_store, do_gather):
        if do_gather:
            prep_idx(u + 1, 1 - p)
            start_gather(1 - p)
        wait_gather(p)
        if do_wait_store:
            wait_stores(u - 2, p)
        transpose(p)
        start_stores(u, p)

    prep_idx(0, 0)
    start_gather(0)
    body(0, 0, False, True)
    body(1, 1, False, True)

    @pl.loop(2, N_UNITS - 2, step=2)
    def _steady(u0):
        body(u0, 0, True, True)
        body(u0 + 1, 1, True, True)

    body(N_UNITS - 2, 0, True, True)
    body(N_UNITS - 1, 1, True, False)

    wait_stores(N_UNITS - 2, 0)
    wait_stores(N_UNITS - 1, 1)


@jax.jit
def _emb(ids_t, weight):
    mesh = plsc.VectorSubcoreMesh(
        core_axis_name="c", subcore_axis_name="s", num_cores=NC, num_subcores=NS
    )
    run = pl.kernel(
        _emb_kernel,
        out_type=jax.ShapeDtypeStruct((H, 8, 128, 1024), jnp.float32),
        mesh=mesh,
        scratch_types=[
            pltpu.VMEM((H, 512), jnp.int32),
            pltpu.VMEM((128,), jnp.int32),
            pltpu.VMEM((128,), jnp.int32),
            pltpu.VMEM((128, DIM), jnp.float32),
            pltpu.VMEM((128, DIM), jnp.float32),
            pltpu.VMEM((8192,), jnp.float32),
            pltpu.VMEM((8192,), jnp.float32),
        ] + [pltpu.SemaphoreType.DMA] * 5,
        compiler_params=pltpu.CompilerParams(use_tc_tiling_on_sc=False,
                                             needs_layout_passes=False),
    )
    return run(ids_t, weight)


def kernel(input_ids, weight):
    ids_t = input_ids.T
    out7 = _emb(ids_t, weight).reshape(H, 8, 128, 8, 128)
    return out7.transpose(2, 4, 0, 1, 3).reshape(B, H, DIM)


# speedup vs baseline: 1.5632x; 1.5631x over previous
import jax
import jax.numpy as jnp
from jax import lax
from jax.experimental import pallas as pl
from jax.experimental.pallas import tpu as pltpu
from jax.experimental.pallas import tpu_sc as plsc

NUM_EMB = 1000000
DIM = 64
B = 16384
H = 50

NC = 2
NS = 16
NW = NC * NS

BGPW = (B // 128) // NW
N_UNITS = H * BGPW


def _emb_kernel(ids_hbm, w_hbm, out_hbm, idsv, idx0, idx1, rows0, rows1,
                tile0, tile1, isem, gsem0, gsem1, tsem0, tsem1):
    wid = lax.axis_index("s") * NC + lax.axis_index("c")
    idxb = (idx0, idx1)
    rows = (rows0, rows1)
    tile = (tile0, tile1)
    gsem = (gsem0, gsem1)
    tsem = (tsem0, tsem1)

    pltpu.async_copy(ids_hbm.at[:, pl.ds(wid * 512, 512)], idsv, isem).wait()

    lane = lax.iota(jnp.int32, 16)

    def prep_idx(u, p):
        h = u // BGPW
        bgl = u % BGPW
        for k in range(8):
            idxb[p][pl.ds(k * 16, 16)] = idsv[h, pl.ds(bgl * 128 + k * 16, 16)]

    def start_gather(p):
        pltpu.async_copy(w_hbm.at[idxb[p]], rows[p], gsem[p])

    def wait_gather(p):
        pltpu.make_async_copy(w_hbm.at[idxb[p]], rows[p], gsem[p]).wait()

    perm = [jnp.mod(lane + j, 16) for j in range(16)]
    dstc = [lane * 128 + perm[j] for j in range(16)]

    def transpose(p):
        @pl.loop(0, 32)
        def _sub(st):
            b0 = (st // 4) * 16
            d0 = (st % 4) * 16
            for j in range(16):
                v = plsc.load_gather(rows[p], [b0 + perm[j], d0 + lane])
                plsc.store_scatter(tile[p], [dstc[j] + (d0 * 128 + b0)], v)

    def start_stores(u, p):
        h = u // BGPW
        bg = wid * BGPW + (u % BGPW)
        for dg in range(8):
            pltpu.async_copy(tile[p].at[pl.ds(dg * 1024, 1024)],
                             out_hbm.at[h, dg, bg], tsem[p])

    def wait_stores(u, p):
        h = u // BGPW
        bg = wid * BGPW + (u % BGPW)
        for dg in range(8):
            pltpu.make_async_copy(tile[p].at[pl.ds(dg * 1024, 1024)],
                                  out_hbm.at[h, dg, bg], tsem[p]).wait()

    def body(u, p, do_wait_store, do_gather):
        if do_gather:
            prep_idx(u + 1, 1 - p)
            start_gather(1 - p)
        wait_gather(p)
        if do_wait_store:
            wait_stores(u - 2, p)
        transpose(p)
        start_stores(u, p)

    prep_idx(0, 0)
    start_gather(0)
    body(0, 0, False, True)
    body(1, 1, False, True)

    @pl.loop(2, N_UNITS - 2, step=2)
    def _steady(u0):
        body(u0, 0, True, True)
        body(u0 + 1, 1, True, True)

    body(N_UNITS - 2, 0, True, True)
    body(N_UNITS - 1, 1, True, False)

    wait_stores(N_UNITS - 2, 0)
    wait_stores(N_UNITS - 1, 1)


@jax.jit
def _emb(ids_t, weight):
    mesh = plsc.VectorSubcoreMesh(
        core_axis_name="c", subcore_axis_name="s", num_cores=NC, num_subcores=NS
    )
    run = pl.kernel(
        _emb_kernel,
        out_type=jax.ShapeDtypeStruct((H, 8, 128, 1024), jnp.float32),
        mesh=mesh,
        scratch_types=[
            pltpu.VMEM((H, 512), jnp.int32),
            pltpu.VMEM((128,), jnp.int32),
            pltpu.VMEM((128,), jnp.int32),
            pltpu.VMEM((128, DIM), jnp.float32),
            pltpu.VMEM((128, DIM), jnp.float32),
            pltpu.VMEM((8192,), jnp.float32),
            pltpu.VMEM((8192,), jnp.float32),
        ] + [pltpu.SemaphoreType.DMA] * 5,
        compiler_params=pltpu.CompilerParams(use_tc_tiling_on_sc=False,
                                             needs_layout_passes=False),
    )
    return run(ids_t, weight)


def kernel(input_ids, weight):
    ids_t = input_ids.T
    out7 = _emb(ids_t, weight).reshape(H, 8, 128, 8, 128)
    return out7.transpose(2, 4, 0, 1, 3).reshape(B, H, DIM)


# speedup vs baseline: 1.9516x; 1.2485x over previous
import jax
import jax.numpy as jnp
from jax import lax
from jax.experimental import pallas as pl
from jax.experimental.pallas import tpu as pltpu
from jax.experimental.pallas import tpu_sc as plsc

NUM_EMB = 1000000
DIM = 64
B = 16384
H = 50

NC = 2
NS = 16
NW = NC * NS

BGPW = (B // 128) // NW
N_UNITS = H * BGPW


def _emb_kernel(ids_hbm, w_hbm, out_hbm, idsv, idx0, idx1, rows0, rows1,
                tile0, tile1, isem, gsem0, gsem1, tsem0, tsem1):
    wid = lax.axis_index("s") * NC + lax.axis_index("c")
    idxb = (idx0, idx1)
    rows = (rows0, rows1)
    tile = (tile0, tile1)
    gsem = (gsem0, gsem1)
    tsem = (tsem0, tsem1)

    pltpu.async_copy(ids_hbm.at[:, pl.ds(wid * 512, 512)], idsv, isem).wait()

    lane = lax.iota(jnp.int32, 16)

    def prep_idx(u, p):
        h = u // BGPW
        bgl = u % BGPW
        for k in range(8):
            idxb[p][pl.ds(k * 16, 16)] = idsv[h, pl.ds(bgl * 128 + k * 16, 16)]

    def start_gather(p):
        pltpu.async_copy(w_hbm.at[idxb[p]], rows[p], gsem[p])

    def wait_gather(p):
        pltpu.make_async_copy(w_hbm.at[idxb[p]], rows[p], gsem[p]).wait()

    perm = [jnp.mod(lane + j, 16) for j in range(16)]
    dstc = [lane * 128 + perm[j] for j in range(16)]

    def transpose(p):
        @pl.loop(0, 32, unroll=2)
        def _sub(st):
            b0 = (st // 4) * 16
            d0 = (st % 4) * 16
            dl = d0 + lane
            sb = d0 * 128 + b0
            vs = [plsc.load_gather(rows[p], [b0 + perm[j], dl])
                  for j in range(16)]
            for j in range(16):
                plsc.store_scatter(tile[p], [dstc[j] + sb], vs[j])

    def start_stores(u, p):
        h = u // BGPW
        bg = wid * BGPW + (u % BGPW)
        for dg in range(8):
            pltpu.async_copy(tile[p].at[pl.ds(dg * 1024, 1024)],
                             out_hbm.at[h, dg, bg], tsem[p])

    def wait_stores(u, p):
        h = u // BGPW
        bg = wid * BGPW + (u % BGPW)
        for dg in range(8):
            pltpu.make_async_copy(tile[p].at[pl.ds(dg * 1024, 1024)],
                                  out_hbm.at[h, dg, bg], tsem[p]).wait()

    def body(u, p, do_wait_store, do_gather):
        if do_gather:
            prep_idx(u + 1, 1 - p)
            start_gather(1 - p)
        wait_gather(p)
        if do_wait_store:
            wait_stores(u - 2, p)
        transpose(p)
        start_stores(u, p)

    prep_idx(0, 0)
    start_gather(0)
    body(0, 0, False, True)
    body(1, 1, False, True)

    @pl.loop(2, N_UNITS - 2, step=2)
    def _steady(u0):
        body(u0, 0, True, True)
        body(u0 + 1, 1, True, True)

    body(N_UNITS - 2, 0, True, True)
    body(N_UNITS - 1, 1, True, False)

    wait_stores(N_UNITS - 2, 0)
    wait_stores(N_UNITS - 1, 1)


@jax.jit
def _emb(ids_t, weight):
    mesh = plsc.VectorSubcoreMesh(
        core_axis_name="c", subcore_axis_name="s", num_cores=NC, num_subcores=NS
    )
    run = pl.kernel(
        _emb_kernel,
        out_type=jax.ShapeDtypeStruct((H, 8, 128, 1024), jnp.float32),
        mesh=mesh,
        scratch_types=[
            pltpu.VMEM((H, 512), jnp.int32),
            pltpu.VMEM((128,), jnp.int32),
            pltpu.VMEM((128,), jnp.int32),
            pltpu.VMEM((128, DIM), jnp.float32),
            pltpu.VMEM((128, DIM), jnp.float32),
            pltpu.VMEM((8192,), jnp.float32),
            pltpu.VMEM((8192,), jnp.float32),
        ] + [pltpu.SemaphoreType.DMA] * 5,
        compiler_params=pltpu.CompilerParams(use_tc_tiling_on_sc=False,
                                             needs_layout_passes=False),
    )
    return run(ids_t, weight)


def kernel(input_ids, weight):
    ids_t = input_ids.T
    out7 = _emb(ids_t, weight).reshape(H, 8, 128, 8, 128)
    return out7.transpose(2, 4, 0, 1, 3).reshape(B, H, DIM)


# speedup vs baseline: 3.2507x; 1.6656x over previous
import jax
import jax.numpy as jnp
from jax import lax
from jax.experimental import pallas as pl
from jax.experimental.pallas import tpu as pltpu
from jax.experimental.pallas import tpu_sc as plsc

NUM_EMB = 1000000
DIM = 64
B = 16384
H = 50

NC = 2
NS = 16
NW = NC * NS

BGPW = (B // 128) // NW
N_UNITS = H * BGPW


def _emb_kernel(ids_hbm, w_hbm, out_hbm, idsv, idx0, idx1, rows0, rows1,
                tile0, tile1, isem, gsem0, gsem1, tsem0, tsem1):
    wid = lax.axis_index("s") * NC + lax.axis_index("c")
    idxb = (idx0, idx1)
    rows = (rows0, rows1)
    tile = (tile0, tile1)
    gsem = (gsem0, gsem1)
    tsem = (tsem0, tsem1)

    pltpu.async_copy(ids_hbm.at[:, pl.ds(wid * 512, 512)], idsv, isem).wait()

    lane = lax.iota(jnp.int32, 16)

    def prep_idx(u, p):
        h = u // BGPW
        bgl = u % BGPW
        for k in range(8):
            idxb[p][pl.ds(k * 16, 16)] = idsv[h, pl.ds(bgl * 128 + k * 16, 16)]

    def start_gather(p):
        pltpu.async_copy(w_hbm.at[idxb[p]], rows[p], gsem[p])

    def wait_gather(p):
        pltpu.make_async_copy(w_hbm.at[idxb[p]], rows[p], gsem[p]).wait()

    perm = [jnp.mod(lane + j, 16) for j in range(16)]
    dstc = [lane * 128 + perm[j] for j in range(16)]

    def transpose(p):
        @pl.loop(0, 32, unroll=2)
        def _sub(st):
            b0 = (st // 4) * 16
            d0 = (st % 4) * 16
            dl = d0 + lane
            sb = d0 * 128 + b0
            vs = [plsc.load_gather(rows[p], [b0 + perm[j], dl])
                  for j in range(16)]
            for j in range(16):
                plsc.store_scatter(tile[p], [dstc[j] + sb], vs[j])

    def start_stores(u, p):
        h = u // BGPW
        bg = wid * BGPW + (u % BGPW)
        for dg in range(8):
            pltpu.async_copy(tile[p].at[pl.ds(dg * 1024, 1024)],
                             out_hbm.at[h, dg, bg], tsem[p])

    def wait_stores(u, p):
        h = u // BGPW
        bg = wid * BGPW + (u % BGPW)
        for dg in range(8):
            pltpu.make_async_copy(tile[p].at[pl.ds(dg * 1024, 1024)],
                                  out_hbm.at[h, dg, bg], tsem[p]).wait()

    def body(u, p, do_wait_store, do_gather):
        if do_gather:
            prep_idx(u + 1, 1 - p)
            start_gather(1 - p)
        wait_gather(p)
        if do_wait_store:
            wait_stores(u - 2, p)
        transpose(p)
        start_stores(u, p)

    prep_idx(0, 0)
    start_gather(0)
    body(0, 0, False, True)
    body(1, 1, False, True)

    @pl.loop(2, N_UNITS - 2, step=2)
    def _steady(u0):
        body(u0, 0, True, True)
        body(u0 + 1, 1, True, True)

    body(N_UNITS - 2, 0, True, True)
    body(N_UNITS - 1, 1, True, False)

    wait_stores(N_UNITS - 2, 0)
    wait_stores(N_UNITS - 1, 1)


N_RT = NUM_EMB // 128
RT_PW = 245


def _wtr_kernel(wt_hbm, out_hbm, tv0, tv1, buf0, buf1, tvq, vsem0, vsem1,
                bsem0, bsem1):
    wid = lax.axis_index("s") * NC + lax.axis_index("c")
    tv = (tv0, tv1)
    buf = (buf0, buf1)
    vsem = (vsem0, vsem1)
    bsem = (bsem0, bsem1)
    s = wid * 244 + jnp.minimum(wid, 3)

    lane = lax.iota(jnp.int32, 16)
    perm = [jnp.mod(lane + j, 16) for j in range(16)]
    dstc = [lane * 64 + perm[j] for j in range(16)]

    def start_in(t, p):
        for g in range(8):
            pltpu.async_copy(wt_hbm.at[pl.ds(g * 8, 8), pl.ds(t * 128, 128)],
                             tv[p].at[pl.ds(g * 8, 8)], vsem[p])

    def wait_in(t, p):
        for g in range(8):
            pltpu.make_async_copy(wt_hbm.at[pl.ds(g * 8, 8), pl.ds(t * 128, 128)],
                                  tv[p].at[pl.ds(g * 8, 8)], vsem[p]).wait()

    def transpose(p):
        @pl.loop(0, 32, unroll=2)
        def _sub(st):
            d0 = (st // 8) * 16
            rr0 = (st % 8) * 16
            dl = rr0 + lane
            sb = rr0 * 64 + d0
            vs = [plsc.load_gather(tv[p], [d0 + perm[j], dl])
                  for j in range(16)]
            for j in range(16):
                plsc.store_scatter(buf[p], [dstc[j] + sb], vs[j])

    def start_out(t, p):
        pltpu.async_copy(buf[p], out_hbm.at[pl.ds(t * 8192, 8192)], bsem[p])

    def wait_out(t, p):
        pltpu.make_async_copy(buf[p], out_hbm.at[pl.ds(t * 8192, 8192)],
                              bsem[p]).wait()

    def body(i, p, do_wait_out, do_in):
        t = s + i
        if do_in:
            start_in(s + i + 1, 1 - p)
        wait_in(t, p)
        if do_wait_out:
            wait_out(s + i - 2, p)
        transpose(p)
        start_out(t, p)

    start_in(s, 0)
    body(0, 0, False, True)
    body(1, 1, False, True)
    body(2, 0, True, True)

    @pl.loop(3, RT_PW - 2, step=2)
    def _steady(i0):
        body(i0, 1, True, True)
        body(i0 + 1, 0, True, True)

    body(RT_PW - 2, 1, True, True)
    body(RT_PW - 1, 0, True, False)
    wait_out(s + RT_PW - 2, 1)
    wait_out(s + RT_PW - 1, 0)

    @pl.when(wid == NW - 1)
    def _tail():
        for g in range(8):
            pltpu.async_copy(wt_hbm.at[pl.ds(g * 8, 8), pl.ds(N_RT * 128, 64)],
                             tvq.at[pl.ds(g * 8, 8)], vsem0)
        for g in range(8):
            pltpu.make_async_copy(wt_hbm.at[pl.ds(g * 8, 8), pl.ds(N_RT * 128, 64)],
                                  tvq.at[pl.ds(g * 8, 8)], vsem0).wait()
        @pl.loop(0, 16)
        def _subq(st):
            d0 = (st // 4) * 16
            rr0 = (st % 4) * 16
            dl = rr0 + lane
            sb = rr0 * 64 + d0
            vs = [plsc.load_gather(tvq, [d0 + perm[j], dl])
                  for j in range(16)]
            for j in range(16):
                plsc.store_scatter(buf0, [dstc[j] + sb], vs[j])
        pltpu.async_copy(buf0.at[pl.ds(0, 4096)],
                         out_hbm.at[pl.ds(N_RT * 8192, 4096)], bsem0)
        pltpu.make_async_copy(buf0.at[pl.ds(0, 4096)],
                              out_hbm.at[pl.ds(N_RT * 8192, 4096)], bsem0).wait()


@jax.jit
def _wtr(wt):
    mesh = plsc.VectorSubcoreMesh(
        core_axis_name="c", subcore_axis_name="s", num_cores=NC, num_subcores=NS
    )
    run = pl.kernel(
        _wtr_kernel,
        out_type=jax.ShapeDtypeStruct((NUM_EMB * DIM,), jnp.float32),
        mesh=mesh,
        scratch_types=[
            pltpu.VMEM((DIM, 128), jnp.float32),
            pltpu.VMEM((DIM, 128), jnp.float32),
            pltpu.VMEM((8192,), jnp.float32),
            pltpu.VMEM((8192,), jnp.float32),
            pltpu.VMEM((DIM, 64), jnp.float32),
        ] + [pltpu.SemaphoreType.DMA] * 4,
        compiler_params=pltpu.CompilerParams(use_tc_tiling_on_sc=True,
                                             needs_layout_passes=False),
    )
    return run(wt)


@jax.jit
def _emb(ids_t, weight):
    mesh = plsc.VectorSubcoreMesh(
        core_axis_name="c", subcore_axis_name="s", num_cores=NC, num_subcores=NS
    )
    run = pl.kernel(
        _emb_kernel,
        out_type=jax.ShapeDtypeStruct((H, 8, 128, 1024), jnp.float32),
        mesh=mesh,
        scratch_types=[
            pltpu.VMEM((H, 512), jnp.int32),
            pltpu.VMEM((128,), jnp.int32),
            pltpu.VMEM((128,), jnp.int32),
            pltpu.VMEM((128, DIM), jnp.float32),
            pltpu.VMEM((128, DIM), jnp.float32),
            pltpu.VMEM((8192,), jnp.float32),
            pltpu.VMEM((8192,), jnp.float32),
        ] + [pltpu.SemaphoreType.DMA] * 5,
        compiler_params=pltpu.CompilerParams(use_tc_tiling_on_sc=False,
                                             needs_layout_passes=False),
    )
    return run(ids_t, weight)


def kernel(input_ids, weight):
    ids_t = input_ids.T
    wlin = _wtr(weight.T)
    out7 = _emb(ids_t, wlin.reshape(NUM_EMB, DIM)).reshape(H, 8, 128, 8, 128)
    return out7.transpose(2, 4, 0, 1, 3).reshape(B, H, DIM)


# speedup vs baseline: 3.5162x; 1.0817x over previous
import jax
import jax.numpy as jnp
from jax import lax
from jax.experimental import pallas as pl
from jax.experimental.pallas import tpu as pltpu
from jax.experimental.pallas import tpu_sc as plsc

NUM_EMB = 1000000
DIM = 64
B = 16384
H = 50

NC = 2
NS = 16
NW = NC * NS

BGPW = (B // 128) // NW
N_UNITS = H * BGPW


def _emb_kernel(ids_hbm, w_hbm, out_hbm, idsv, idx0, idx1, rows0, rows1,
                tile0, tile1, isem, gsem0, gsem1, tsem0, tsem1):
    wid = lax.axis_index("s") * NC + lax.axis_index("c")
    idxb = (idx0, idx1)
    rows = (rows0, rows1)
    tile = (tile0, tile1)
    gsem = (gsem0, gsem1)
    tsem = (tsem0, tsem1)

    pltpu.async_copy(ids_hbm.at[:, pl.ds(wid * 512, 512)], idsv, isem).wait()

    lane = lax.iota(jnp.int32, 16)

    def prep_idx(u, p):
        h = u // BGPW
        bgl = u % BGPW
        for k in range(8):
            idxb[p][pl.ds(k * 16, 16)] = idsv[h, pl.ds(bgl * 128 + k * 16, 16)]

    def start_gather(p):
        pltpu.async_copy(w_hbm.at[idxb[p]], rows[p], gsem[p])

    def wait_gather(p):
        pltpu.make_async_copy(w_hbm.at[idxb[p]], rows[p], gsem[p]).wait()

    perm = [jnp.mod(lane + j, 16) for j in range(16)]
    dstc = [lane * 128 + perm[j] for j in range(16)]

    def transpose(p):
        @pl.loop(0, 32, unroll=4)
        def _sub(st):
            b0 = (st // 4) * 16
            d0 = (st % 4) * 16
            dl = d0 + lane
            sb = d0 * 128 + b0
            vs = [plsc.load_gather(rows[p], [b0 + perm[j], dl])
                  for j in range(16)]
            for j in range(16):
                plsc.store_scatter(tile[p], [dstc[j] + sb], vs[j])

    def start_stores(u, p):
        h = u // BGPW
        bg = wid * BGPW + (u % BGPW)
        for dg in range(8):
            pltpu.async_copy(tile[p].at[pl.ds(dg * 1024, 1024)],
                             out_hbm.at[h, dg, bg], tsem[p])

    def wait_stores(u, p):
        h = u // BGPW
        bg = wid * BGPW + (u % BGPW)
        for dg in range(8):
            pltpu.make_async_copy(tile[p].at[pl.ds(dg * 1024, 1024)],
                                  out_hbm.at[h, dg, bg], tsem[p]).wait()

    def body(u, p, do_wait_store, do_gather):
        if do_gather:
            prep_idx(u + 1, 1 - p)
            start_gather(1 - p)
        wait_gather(p)
        if do_wait_store:
            wait_stores(u - 2, p)
        transpose(p)
        start_stores(u, p)

    prep_idx(0, 0)
    start_gather(0)
    body(0, 0, False, True)
    body(1, 1, False, True)

    @pl.loop(2, N_UNITS - 2, step=2)
    def _steady(u0):
        body(u0, 0, True, True)
        body(u0 + 1, 1, True, True)

    body(N_UNITS - 2, 0, True, True)
    body(N_UNITS - 1, 1, True, False)

    wait_stores(N_UNITS - 2, 0)
    wait_stores(N_UNITS - 1, 1)


N_RT = NUM_EMB // 128
RT_PW = 245


def _wtr_kernel(wt_hbm, out_hbm, tv0, tv1, buf0, buf1, tvq, vsem0, vsem1,
                bsem0, bsem1):
    wid = lax.axis_index("s") * NC + lax.axis_index("c")
    tv = (tv0, tv1)
    buf = (buf0, buf1)
    vsem = (vsem0, vsem1)
    bsem = (bsem0, bsem1)
    s = wid * 244 + jnp.minimum(wid, 3)

    lane = lax.iota(jnp.int32, 16)
    perm = [jnp.mod(lane + j, 16) for j in range(16)]
    dstc = [lane * 64 + perm[j] for j in range(16)]

    def start_in(t, p):
        pltpu.async_copy(wt_hbm.at[:, pl.ds(t * 128, 128)], tv[p], vsem[p])

    def wait_in(t, p):
        pltpu.make_async_copy(wt_hbm.at[:, pl.ds(t * 128, 128)], tv[p],
                              vsem[p]).wait()

    def transpose(p):
        @pl.loop(0, 32, unroll=4)
        def _sub(st):
            d0 = (st // 8) * 16
            rr0 = (st % 8) * 16
            dl = rr0 + lane
            sb = rr0 * 64 + d0
            vs = [plsc.load_gather(tv[p], [d0 + perm[j], dl])
                  for j in range(16)]
            for j in range(16):
                plsc.store_scatter(buf[p], [dstc[j] + sb], vs[j])

    def start_out(t, p):
        pltpu.async_copy(buf[p], out_hbm.at[pl.ds(t * 8192, 8192)], bsem[p])

    def wait_out(t, p):
        pltpu.make_async_copy(buf[p], out_hbm.at[pl.ds(t * 8192, 8192)],
                              bsem[p]).wait()

    def body(i, p, do_wait_out, do_in):
        t = s + i
        if do_in:
            start_in(s + i + 1, 1 - p)
        wait_in(t, p)
        if do_wait_out:
            wait_out(s + i - 2, p)
        transpose(p)
        start_out(t, p)

    start_in(s, 0)
    body(0, 0, False, True)
    body(1, 1, False, True)
    body(2, 0, True, True)

    @pl.loop(3, RT_PW - 2, step=2)
    def _steady(i0):
        body(i0, 1, True, True)
        body(i0 + 1, 0, True, True)

    body(RT_PW - 2, 1, True, True)
    body(RT_PW - 1, 0, True, False)
    wait_out(s + RT_PW - 2, 1)
    wait_out(s + RT_PW - 1, 0)

    @pl.when(wid == NW - 1)
    def _tail():
        for g in range(8):
            pltpu.async_copy(wt_hbm.at[pl.ds(g * 8, 8), pl.ds(N_RT * 128, 64)],
                             tvq.at[pl.ds(g * 8, 8)], vsem0)
        for g in range(8):
            pltpu.make_async_copy(wt_hbm.at[pl.ds(g * 8, 8), pl.ds(N_RT * 128, 64)],
                                  tvq.at[pl.ds(g * 8, 8)], vsem0).wait()
        @pl.loop(0, 16)
        def _subq(st):
            d0 = (st // 4) * 16
            rr0 = (st % 4) * 16
            dl = rr0 + lane
            sb = rr0 * 64 + d0
            vs = [plsc.load_gather(tvq, [d0 + perm[j], dl])
                  for j in range(16)]
            for j in range(16):
                plsc.store_scatter(buf0, [dstc[j] + sb], vs[j])
        pltpu.async_copy(buf0.at[pl.ds(0, 4096)],
                         out_hbm.at[pl.ds(N_RT * 8192, 4096)], bsem0)
        pltpu.make_async_copy(buf0.at[pl.ds(0, 4096)],
                              out_hbm.at[pl.ds(N_RT * 8192, 4096)], bsem0).wait()


@jax.jit
def _wtr(wt):
    mesh = plsc.VectorSubcoreMesh(
        core_axis_name="c", subcore_axis_name="s", num_cores=NC, num_subcores=NS
    )
    run = pl.kernel(
        _wtr_kernel,
        out_type=jax.ShapeDtypeStruct((NUM_EMB * DIM,), jnp.float32),
        mesh=mesh,
        scratch_types=[
            pltpu.VMEM((DIM, 128), jnp.float32),
            pltpu.VMEM((DIM, 128), jnp.float32),
            pltpu.VMEM((8192,), jnp.float32),
            pltpu.VMEM((8192,), jnp.float32),
            pltpu.VMEM((DIM, 64), jnp.float32),
        ] + [pltpu.SemaphoreType.DMA] * 4,
        compiler_params=pltpu.CompilerParams(use_tc_tiling_on_sc=True,
                                             needs_layout_passes=False),
    )
    return run(wt)


@jax.jit
def _emb(ids_t, weight):
    mesh = plsc.VectorSubcoreMesh(
        core_axis_name="c", subcore_axis_name="s", num_cores=NC, num_subcores=NS
    )
    run = pl.kernel(
        _emb_kernel,
        out_type=jax.ShapeDtypeStruct((H, 8, 128, 1024), jnp.float32),
        mesh=mesh,
        scratch_types=[
            pltpu.VMEM((H, 512), jnp.int32),
            pltpu.VMEM((128,), jnp.int32),
            pltpu.VMEM((128,), jnp.int32),
            pltpu.VMEM((128, DIM), jnp.float32),
            pltpu.VMEM((128, DIM), jnp.float32),
            pltpu.VMEM((8192,), jnp.float32),
            pltpu.VMEM((8192,), jnp.float32),
        ] + [pltpu.SemaphoreType.DMA] * 5,
        compiler_params=pltpu.CompilerParams(use_tc_tiling_on_sc=False,
                                             needs_layout_passes=False),
    )
    return run(ids_t, weight)


def kernel(input_ids, weight):
    ids_t = input_ids.T
    wlin = _wtr(weight.T)
    out7 = _emb(ids_t, wlin.reshape(NUM_EMB, DIM)).reshape(H, 8, 128, 8, 128)
    return out7.transpose(2, 4, 0, 1, 3).reshape(B, H, DIM)


# speedup vs baseline: 3.8490x; 1.0946x over previous
import jax
import jax.numpy as jnp
from jax import lax
from jax.experimental import pallas as pl
from jax.experimental.pallas import tpu as pltpu
from jax.experimental.pallas import tpu_sc as plsc

NUM_EMB = 1000000
DIM = 64
B = 16384
H = 50

NC = 2
NS = 16
NW = NC * NS

BGPW = (B // 128) // NW
N_UNITS = H * BGPW


def _emb_kernel(ids_hbm, w_hbm, out_hbm, idsv, idx0, idx1, rows0, rows1,
                tile0, tile1, isem, gsem0, gsem1, tsem0, tsem1):
    wid = lax.axis_index("s") * NC + lax.axis_index("c")
    idxb = (idx0, idx1)
    rows = (rows0, rows1)
    tile = (tile0, tile1)
    gsem = (gsem0, gsem1)
    tsem = (tsem0, tsem1)

    pltpu.async_copy(ids_hbm.at[:, pl.ds(wid * 512, 512)], idsv, isem).wait()

    lane = lax.iota(jnp.int32, 16)

    def prep_idx(u, p):
        h = u // BGPW
        bgl = u % BGPW
        for k in range(8):
            idxb[p][pl.ds(k * 16, 16)] = idsv[h, pl.ds(bgl * 128 + k * 16, 16)]

    def start_gather(p):
        pltpu.async_copy(w_hbm.at[idxb[p]], rows[p], gsem[p])

    def wait_gather(p):
        pltpu.make_async_copy(w_hbm.at[idxb[p]], rows[p], gsem[p]).wait()

    perm = [jnp.mod(lane + j, 16) for j in range(16)]
    dstc = [lane * 128 + perm[j] for j in range(16)]

    def transpose(p):
        @pl.loop(0, 32, unroll=4)
        def _sub(st):
            b0 = (st // 4) * 16
            d0 = (st % 4) * 16
            dl = d0 + lane
            sb = d0 * 128 + b0
            vs = [plsc.load_gather(rows[p], [b0 + perm[j], dl])
                  for j in range(16)]
            for j in range(16):
                plsc.store_scatter(tile[p], [dstc[j] + sb], vs[j])

    def start_stores(u, p):
        h = u // BGPW
        bg = wid * BGPW + (u % BGPW)
        for dg in range(8):
            pltpu.async_copy(tile[p].at[pl.ds(dg * 1024, 1024)],
                             out_hbm.at[h, dg, bg], tsem[p])

    def wait_stores(u, p):
        h = u // BGPW
        bg = wid * BGPW + (u % BGPW)
        for dg in range(8):
            pltpu.make_async_copy(tile[p].at[pl.ds(dg * 1024, 1024)],
                                  out_hbm.at[h, dg, bg], tsem[p]).wait()

    def body(u, p, do_wait_store, do_gather):
        if do_gather:
            prep_idx(u + 1, 1 - p)
            start_gather(1 - p)
        wait_gather(p)
        if do_wait_store:
            wait_stores(u - 2, p)
        transpose(p)
        start_stores(u, p)

    prep_idx(0, 0)
    start_gather(0)
    body(0, 0, False, True)
    body(1, 1, False, True)

    @pl.loop(2, N_UNITS - 2, step=2)
    def _steady(u0):
        body(u0, 0, True, True)
        body(u0 + 1, 1, True, True)

    body(N_UNITS - 2, 0, True, True)
    body(N_UNITS - 1, 1, True, False)

    wait_stores(N_UNITS - 2, 0)
    wait_stores(N_UNITS - 1, 1)


N_RT = NUM_EMB // 128
N_RT2 = NUM_EMB // 256
RT_PW = 123


def _wtr_kernel(wt_hbm, out_hbm, tv0, tv1, buf0, buf1, tvq, vsem0, vsem1,
                bsem0, bsem1):
    wid = lax.axis_index("s") * NC + lax.axis_index("c")
    tv = (tv0, tv1)
    buf = (buf0, buf1)
    vsem = (vsem0, vsem1)
    bsem = (bsem0, bsem1)
    s = wid * 122 + jnp.minimum(wid, 1)

    lane = lax.iota(jnp.int32, 16)
    perm = [jnp.mod(lane + j, 16) for j in range(16)]
    dstc = [lane * 64 + perm[j] for j in range(16)]

    def start_in(t, p):
        pltpu.async_copy(wt_hbm.at[:, pl.ds(t * 256, 256)], tv[p], vsem[p])

    def wait_in(t, p):
        pltpu.make_async_copy(wt_hbm.at[:, pl.ds(t * 256, 256)], tv[p],
                              vsem[p]).wait()

    def transpose(p):
        @pl.loop(0, 64, unroll=4)
        def _sub(st):
            d0 = (st // 16) * 16
            rr0 = (st % 16) * 16
            dl = rr0 + lane
            sb = rr0 * 64 + d0
            vs = [plsc.load_gather(tv[p], [d0 + perm[j], dl])
                  for j in range(16)]
            for j in range(16):
                plsc.store_scatter(buf[p], [dstc[j] + sb], vs[j])

    def start_out(t, p):
        pltpu.async_copy(buf[p], out_hbm.at[pl.ds(t * 16384, 16384)], bsem[p])

    def wait_out(t, p):
        pltpu.make_async_copy(buf[p], out_hbm.at[pl.ds(t * 16384, 16384)],
                              bsem[p]).wait()

    def body(i, p, do_wait_out, do_in):
        t = s + i
        if do_in:
            start_in(s + i + 1, 1 - p)
        wait_in(t, p)
        if do_wait_out:
            wait_out(s + i - 2, p)
        transpose(p)
        start_out(t, p)

    start_in(s, 0)
    body(0, 0, False, True)
    body(1, 1, False, True)
    body(2, 0, True, True)

    @pl.loop(3, RT_PW - 2, step=2)
    def _steady(i0):
        body(i0, 1, True, True)
        body(i0 + 1, 0, True, True)

    body(RT_PW - 2, 1, True, True)
    body(RT_PW - 1, 0, True, False)
    wait_out(s + RT_PW - 2, 1)
    wait_out(s + RT_PW - 1, 0)

    @pl.when(wid == NW - 1)
    def _tail():
        for g in range(8):
            pltpu.async_copy(wt_hbm.at[pl.ds(g * 8, 8), pl.ds(N_RT * 128, 64)],
                             tvq.at[pl.ds(g * 8, 8)], vsem0)
        for g in range(8):
            pltpu.make_async_copy(wt_hbm.at[pl.ds(g * 8, 8), pl.ds(N_RT * 128, 64)],
                                  tvq.at[pl.ds(g * 8, 8)], vsem0).wait()
        @pl.loop(0, 16)
        def _subq(st):
            d0 = (st // 4) * 16
            rr0 = (st % 4) * 16
            dl = rr0 + lane
            sb = rr0 * 64 + d0
            vs = [plsc.load_gather(tvq, [d0 + perm[j], dl])
                  for j in range(16)]
            for j in range(16):
                plsc.store_scatter(buf0, [dstc[j] + sb], vs[j])
        pltpu.async_copy(buf0.at[pl.ds(0, 4096)],
                         out_hbm.at[pl.ds(N_RT * 8192, 4096)], bsem0)
        pltpu.make_async_copy(buf0.at[pl.ds(0, 4096)],
                              out_hbm.at[pl.ds(N_RT * 8192, 4096)], bsem0).wait()


@jax.jit
def _wtr(wt):
    mesh = plsc.VectorSubcoreMesh(
        core_axis_name="c", subcore_axis_name="s", num_cores=NC, num_subcores=NS
    )
    run = pl.kernel(
        _wtr_kernel,
        out_type=jax.ShapeDtypeStruct((NUM_EMB * DIM,), jnp.float32),
        mesh=mesh,
        scratch_types=[
            pltpu.VMEM((DIM, 256), jnp.float32),
            pltpu.VMEM((DIM, 256), jnp.float32),
            pltpu.VMEM((16384,), jnp.float32),
            pltpu.VMEM((16384,), jnp.float32),
            pltpu.VMEM((DIM, 64), jnp.float32),
        ] + [pltpu.SemaphoreType.DMA] * 4,
        compiler_params=pltpu.CompilerParams(use_tc_tiling_on_sc=True,
                                             needs_layout_passes=False),
    )
    return run(wt)


@jax.jit
def _emb(ids_t, weight):
    mesh = plsc.VectorSubcoreMesh(
        core_axis_name="c", subcore_axis_name="s", num_cores=NC, num_subcores=NS
    )
    run = pl.kernel(
        _emb_kernel,
        out_type=jax.ShapeDtypeStruct((H, 8, 128, 1024), jnp.float32),
        mesh=mesh,
        scratch_types=[
            pltpu.VMEM((H, 512), jnp.int32),
            pltpu.VMEM((128,), jnp.int32),
            pltpu.VMEM((128,), jnp.int32),
            pltpu.VMEM((128, DIM), jnp.float32),
            pltpu.VMEM((128, DIM), jnp.float32),
            pltpu.VMEM((8192,), jnp.float32),
            pltpu.VMEM((8192,), jnp.float32),
        ] + [pltpu.SemaphoreType.DMA] * 5,
        compiler_params=pltpu.CompilerParams(use_tc_tiling_on_sc=False,
                                             needs_layout_passes=False),
    )
    return run(ids_t, weight)


def kernel(input_ids, weight):
    ids_t = input_ids.T
    wlin = _wtr(weight.T)
    out7 = _emb(ids_t, wlin.reshape(NUM_EMB, DIM)).reshape(H, 8, 128, 8, 128)
    return out7.transpose(2, 4, 0, 1, 3).reshape(B, H, DIM)
